# baseline (speedup 1.0000x reference)
import functools

import jax
import jax.numpy as jnp
from jax import lax
from jax.experimental import pallas as pl
from jax.experimental.pallas import tpu as pltpu
from jax.experimental.pallas import tpu_sc as plsc

_NC = 2
_NS = 16
_NW = _NC * _NS
_CHUNK = 128
_F = 16


def _mesh():
    return plsc.VectorSubcoreMesh(core_axis_name="c", subcore_axis_name="s")


_TRIPS = 80


def _zero_acc(stage_v, acc_sh, sid, rpt):
    def fill_zero(i, c):
        stage_v[i] = jnp.zeros((_F,), jnp.float32)
        return c

    lax.fori_loop(0, rpt, fill_zero, 0)
    pltpu.sync_copy(stage_v, acc_sh.at[pl.ds(sid * rpt, rpt)])
    plsc.subcore_barrier()


def _copy_out(stage_v, acc_sh, out_hbm, cid, sid, rpt):
    plsc.subcore_barrier()
    pltpu.sync_copy(acc_sh.at[pl.ds(sid * rpt, rpt)], stage_v)
    pltpu.sync_copy(stage_v, out_hbm.at[cid, pl.ds(sid * rpt, rpt)])


def _sc_count(dst3, n_pad):
    rpt = n_pad // _NS

    @functools.partial(
        pl.kernel,
        mesh=_mesh(),
        out_type=jax.ShapeDtypeStruct((_NC, n_pad, _F), jnp.float32),
        compiler_params=pltpu.CompilerParams(use_tc_tiling_on_sc=False),
        scratch_types=[
            pltpu.VMEM((_TRIPS, _CHUNK), jnp.int32),
            pltpu.VMEM((_CHUNK, _F), jnp.float32),
            pltpu.VMEM((rpt, _F), jnp.float32),
            pltpu.VMEM_SHARED((n_pad, _F), jnp.float32),
            pltpu.SemaphoreType.DMA,
            pltpu.SemaphoreType.DMA,
            pltpu.SemaphoreType.DMA,
            pltpu.SemaphoreType.DMA,
        ],
    )
    def k(dst_hbm, out_hbm, didx_v, ones_v, stage_v, acc_sh, s0, s1, s2, s3):
        cid = lax.axis_index("c")
        sid = lax.axis_index("s")
        wid = sid * _NC + cid
        sems = [s0, s1, s2, s3]

        def fill_ones(i, c):
            ones_v[i] = jnp.ones((_F,), jnp.float32)
            return c

        lax.fori_loop(0, _CHUNK, fill_ones, 0)
        pltpu.sync_copy(dst_hbm.at[wid, :, :], didx_v)
        _zero_acc(stage_v, acc_sh, sid, rpt)

        def swait(sem):
            pltpu.make_async_copy(ones_v, acc_sh.at[didx_v.at[0]], sem).wait()

        def outer(g4, c):
            g = g4 * 4
            for b in range(4):
                j = g + b

                @pl.when(j >= 4)
                def _():
                    swait(sems[b])

                pltpu.async_copy(ones_v, acc_sh.at[didx_v.at[j]], sems[b],
                                 add=True)
            return c

        lax.fori_loop(0, _TRIPS // 4, outer, 0)
        for b in range(4):
            swait(sems[b])
        _copy_out(stage_v, acc_sh, out_hbm, cid, sid, rpt)

    return k(dst3)


def _sc_segsum(table, src3, dst3, n_pad):
    rpt = n_pad // _NS

    @functools.partial(
        pl.kernel,
        mesh=_mesh(),
        out_type=jax.ShapeDtypeStruct((_NC, n_pad, _F), jnp.float32),
        compiler_params=pltpu.CompilerParams(use_tc_tiling_on_sc=False),
        scratch_types=[
            pltpu.VMEM((_TRIPS, _CHUNK), jnp.int32),
            pltpu.VMEM((_TRIPS, _CHUNK), jnp.int32),
            pltpu.VMEM((_CHUNK, _F), jnp.float32),
            pltpu.VMEM((_CHUNK, _F), jnp.float32),
            pltpu.VMEM((_CHUNK, _F), jnp.float32),
            pltpu.VMEM((_CHUNK, _F), jnp.float32),
            pltpu.VMEM((rpt, _F), jnp.float32),
            pltpu.VMEM_SHARED((n_pad, _F), jnp.float32),
            pltpu.SemaphoreType.DMA,
            pltpu.SemaphoreType.DMA,
            pltpu.SemaphoreType.DMA,
            pltpu.SemaphoreType.DMA,
            pltpu.SemaphoreType.DMA,
            pltpu.SemaphoreType.DMA,
            pltpu.SemaphoreType.DMA,
            pltpu.SemaphoreType.DMA,
        ],
    )
    def k(table_hbm, src_hbm, dst_hbm, out_hbm,
          sidx_v, didx_v, r0, r1, r2, r3, stage_v, acc_sh,
          g0, g1, g2, g3, s0, s1, s2, s3):
        cid = lax.axis_index("c")
        sid = lax.axis_index("s")
        wid = sid * _NC + cid
        rows = [r0, r1, r2, r3]
        gsem = [g0, g1, g2, g3]
        ssem = [s0, s1, s2, s3]

        pltpu.sync_copy(src_hbm.at[wid, :, :], sidx_v)
        pltpu.sync_copy(dst_hbm.at[wid, :, :], didx_v)
        _zero_acc(stage_v, acc_sh, sid, rpt)

        def gwait(b):
            pltpu.make_async_copy(table_hbm.at[sidx_v.at[0]], rows[b],
                                  gsem[b]).wait()

        def swait(b):
            pltpu.make_async_copy(rows[b], acc_sh.at[didx_v.at[0]],
                                  ssem[b]).wait()

        pltpu.async_copy(table_hbm.at[sidx_v.at[0]], rows[0], gsem[0])
        pltpu.async_copy(table_hbm.at[sidx_v.at[1]], rows[1], gsem[1])

        def outer(g4, c):
            g = g4 * 4
            for b in range(4):
                j = g + b
                bf = (b + 2) % 4

                @pl.when(j + 2 < _TRIPS)
                def _():
                    @pl.when(j >= 2)
                    def _():
                        swait(bf)

                    pltpu.async_copy(table_hbm.at[sidx_v.at[j + 2]],
                                     rows[bf], gsem[bf])

                gwait(b)
                pltpu.async_copy(rows[b], acc_sh.at[didx_v.at[j]], ssem[b],
                                 add=True)
            return c

        lax.fori_loop(0, _TRIPS // 4, outer, 0)
        for b in range(4):
            swait(b)
        _copy_out(stage_v, acc_sh, out_hbm, cid, sid, rpt)

    return k(table, src3, dst3)


def _tc_prep(x, W1, cnt_p, n_pad):
    N = x.shape[0]

    def body(x_ref, w_ref, cnt_ref, hx_ref, hxs_ref, dinvb_ref, invcb_ref):
        hx = jnp.dot(x_ref[...], w_ref[...], preferred_element_type=jnp.float32)
        hx_ref[0:N, :] = hx
        hx_ref[N:n_pad, :] = jnp.zeros((n_pad - N, _F), jnp.float32)
        cnt = cnt_ref[0] + cnt_ref[1]
        dinvb = lax.rsqrt(cnt + 1.0)
        dinvb_ref[...] = dinvb
        invcb_ref[...] = 1.0 / jnp.maximum(cnt, 1.0)
        hxs_ref[...] = hx_ref[...] * dinvb

    shp = jax.ShapeDtypeStruct((n_pad, _F), jnp.float32)
    return pl.pallas_call(body, out_shape=(shp, shp, shp, shp))(x, W1, cnt_p)


def _tc_comb(t1_p, hx, dinvb, b1):
    n_pad = hx.shape[0]

    def body(t1_ref, hx_ref, dinvb_ref, b1_ref, h_ref):
        d = dinvb_ref[...]
        t1 = t1_ref[0] + t1_ref[1]
        h_ref[...] = d * t1 + d * d * hx_ref[...] + b1_ref[...][None, :]

    return pl.pallas_call(
        body,
        out_shape=jax.ShapeDtypeStruct((n_pad, _F), jnp.float32),
    )(t1_p, hx, dinvb, b1)


def _tc_out(t2_p, h, invcb, Wl, bl, Wr):
    n_pad = h.shape[0]
    C = Wl.shape[1]

    def body(t2_ref, h_ref, invcb_ref, wl_ref, bl_ref, wr_ref, o_ref):
        mean = (t2_ref[0] + t2_ref[1]) * invcb_ref[...]
        h = h_ref[...]
        o = (jnp.dot(mean, wl_ref[...], preferred_element_type=jnp.float32)
             + jnp.dot(h, wr_ref[...], preferred_element_type=jnp.float32)
             + bl_ref[...][None, :])
        m = jnp.max(o, axis=1, keepdims=True)
        lse = m + jnp.log(jnp.sum(jnp.exp(o - m), axis=1, keepdims=True))
        o_ref[...] = o - lse

    return pl.pallas_call(
        body,
        out_shape=jax.ShapeDtypeStruct((n_pad, C), jnp.float32),
    )(t2_p, h, invcb, Wl, bl, Wr)


def kernel(x, edge_index, W1, b1, Wl, bl, Wr):
    N = x.shape[0]
    E = edge_index.shape[1]
    n_pad = ((N + 255) // 256) * 256
    e_pad = _NW * _TRIPS * _CHUNK
    pad = e_pad - E
    assert 0 <= pad
    idx = jnp.arange(pad, dtype=jnp.int32)
    src3 = jnp.concatenate([edge_index[0], idx % N]).reshape(_NW, _TRIPS, _CHUNK)
    dst3 = jnp.concatenate(
        [edge_index[1], N + idx % (n_pad - N)]).reshape(_NW, _TRIPS, _CHUNK)

    cnt_p = _sc_count(dst3, n_pad)
    hx, hxs, dinvb, invcb = _tc_prep(x, W1, cnt_p, n_pad)
    t1_p = _sc_segsum(hxs, src3, dst3, n_pad)
    h = _tc_comb(t1_p, hx, dinvb, b1)
    t2_p = _sc_segsum(h, src3, dst3, n_pad)
    out = _tc_out(t2_p, h, invcb, Wl, bl, Wr)
    return out[:N]


# speedup vs baseline: 1.1151x; 1.1151x over previous
import functools

import jax
import jax.numpy as jnp
from jax import lax
from jax.experimental import pallas as pl
from jax.experimental.pallas import tpu as pltpu
from jax.experimental.pallas import tpu_sc as plsc

_NC = 2
_NS = 16
_NW = _NC * _NS
_CHUNK = 128
_F = 16
_TPW = 80


def _mesh():
    return plsc.VectorSubcoreMesh(core_axis_name="c", subcore_axis_name="s")


def _zero_acc(stage_v, acc_sh, sid, rpt):
    def fill_zero(i, c):
        stage_v[i] = jnp.zeros((_F,), jnp.float32)
        return c

    lax.fori_loop(0, rpt, fill_zero, 0)
    pltpu.sync_copy(stage_v, acc_sh.at[pl.ds(sid * rpt, rpt)])
    plsc.subcore_barrier()


def _copy_out(stage_v, acc_sh, out_hbm, cid, sid, rpt):
    plsc.subcore_barrier()
    pltpu.sync_copy(acc_sh.at[pl.ds(sid * rpt, rpt)], stage_v)
    pltpu.sync_copy(stage_v, out_hbm.at[cid, pl.ds(sid * rpt, rpt)])


def _tile_span(wid, nchunks):
    last = nchunks - (_NW - 1) * _TPW
    base = wid * _TPW
    trips = jnp.where(wid < _NW - 1, _TPW, last)
    return base, trips


def _load_idx(idx_hbm, idx_v, wid, base, nchunks):
    last = nchunks - (_NW - 1) * _TPW

    @pl.when(wid < _NW - 1)
    def _():
        pltpu.sync_copy(idx_hbm.at[pl.ds(base, _TPW), :], idx_v)

    if last < _TPW:
        @pl.when(wid == _NW - 1)
        def _():
            pltpu.sync_copy(idx_hbm.at[pl.ds((_NW - 1) * _TPW, last), :],
                            idx_v.at[pl.ds(0, last), :])


def _sc_count(dst2, n):
    nchunks = dst2.shape[0]
    rpt = n // _NS

    @functools.partial(
        pl.kernel,
        mesh=_mesh(),
        out_type=jax.ShapeDtypeStruct((_NC, n, _F), jnp.float32),
        compiler_params=pltpu.CompilerParams(use_tc_tiling_on_sc=False),
        scratch_types=[
            pltpu.VMEM((_TPW, _CHUNK), jnp.int32),
            pltpu.VMEM((_CHUNK, _F), jnp.float32),
            pltpu.VMEM((rpt, _F), jnp.float32),
            pltpu.VMEM_SHARED((n, _F), jnp.float32),
            pltpu.SemaphoreType.DMA,
            pltpu.SemaphoreType.DMA,
            pltpu.SemaphoreType.DMA,
            pltpu.SemaphoreType.DMA,
        ],
    )
    def k(dst_hbm, out_hbm, didx_v, ones_v, stage_v, acc_sh, s0, s1, s2, s3):
        cid = lax.axis_index("c")
        sid = lax.axis_index("s")
        wid = sid * _NC + cid
        base, trips = _tile_span(wid, nchunks)
        sems = [s0, s1, s2, s3]

        def fill_ones(i, c):
            ones_v[i] = jnp.ones((_F,), jnp.float32)
            return c

        lax.fori_loop(0, _CHUNK, fill_ones, 0)
        _load_idx(dst_hbm, didx_v, wid, base, nchunks)
        _zero_acc(stage_v, acc_sh, sid, rpt)

        def swait(sem):
            pltpu.make_async_copy(ones_v, acc_sh.at[didx_v.at[0]], sem).wait()

        def step(j, c):
            b = lax.rem(j, 4)

            @pl.when(j >= 4)
            def _():
                for bb in range(4):
                    @pl.when(b == bb)
                    def _():
                        swait(sems[bb])

            for bb in range(4):
                @pl.when(b == bb)
                def _():
                    pltpu.async_copy(ones_v, acc_sh.at[didx_v.at[j]],
                                     sems[bb], add=True)
            return c

        lax.fori_loop(0, trips, step, 0)
        for bb in range(4):
            @pl.when(trips >= bb + 1)
            def _():
                swait(sems[bb])

        _copy_out(stage_v, acc_sh, out_hbm, cid, sid, rpt)

    return k(dst2)


def _sc_segsum(table, src2, dst2, n):
    nchunks = src2.shape[0]
    rpt = n // _NS

    @functools.partial(
        pl.kernel,
        mesh=_mesh(),
        out_type=jax.ShapeDtypeStruct((_NC, n, _F), jnp.float32),
        compiler_params=pltpu.CompilerParams(use_tc_tiling_on_sc=False),
        scratch_types=[
            pltpu.VMEM((_TPW, _CHUNK), jnp.int32),
            pltpu.VMEM((_TPW, _CHUNK), jnp.int32),
            pltpu.VMEM((_CHUNK, _F), jnp.float32),
            pltpu.VMEM((_CHUNK, _F), jnp.float32),
            pltpu.VMEM((_CHUNK, _F), jnp.float32),
            pltpu.VMEM((_CHUNK, _F), jnp.float32),
            pltpu.VMEM((rpt, _F), jnp.float32),
            pltpu.VMEM_SHARED((n, _F), jnp.float32),
            pltpu.VMEM_SHARED((n, _F), jnp.float32),
            pltpu.SemaphoreType.DMA,
            pltpu.SemaphoreType.DMA,
            pltpu.SemaphoreType.DMA,
            pltpu.SemaphoreType.DMA,
            pltpu.SemaphoreType.DMA,
            pltpu.SemaphoreType.DMA,
            pltpu.SemaphoreType.DMA,
            pltpu.SemaphoreType.DMA,
        ],
    )
    def k(table_hbm, src_hbm, dst_hbm, out_hbm,
          sidx_v, didx_v, r0, r1, r2, r3, stage_v, tab_sh, acc_sh,
          g0, g1, g2, g3, s0, s1, s2, s3):
        cid = lax.axis_index("c")
        sid = lax.axis_index("s")
        wid = sid * _NC + cid
        base, trips = _tile_span(wid, nchunks)
        rows = [r0, r1, r2, r3]
        gsem = [g0, g1, g2, g3]
        ssem = [s0, s1, s2, s3]

        _load_idx(src_hbm, sidx_v, wid, base, nchunks)
        _load_idx(dst_hbm, didx_v, wid, base, nchunks)
        pltpu.sync_copy(table_hbm.at[pl.ds(sid * rpt, rpt), :], stage_v)
        pltpu.sync_copy(stage_v, tab_sh.at[pl.ds(sid * rpt, rpt)])
        _zero_acc(stage_v, acc_sh, sid, rpt)

        def gstart(j, b):
            pltpu.async_copy(tab_sh.at[sidx_v.at[j]], rows[b], gsem[b])

        def gwait(b):
            pltpu.make_async_copy(tab_sh.at[sidx_v.at[0]], rows[b],
                                  gsem[b]).wait()

        def sstart(j, b):
            pltpu.async_copy(rows[b], acc_sh.at[didx_v.at[j]], ssem[b],
                             add=True)

        def swait(b):
            pltpu.make_async_copy(rows[b], acc_sh.at[didx_v.at[0]],
                                  ssem[b]).wait()

        @pl.when(trips >= 1)
        def _():
            gstart(0, 0)

        @pl.when(trips >= 2)
        def _():
            gstart(1, 1)

        def step(j, c):
            b = lax.rem(j, 4)
            for bb in range(4):
                @pl.when(b == bb)
                def _():
                    bf = (bb + 2) % 4

                    @pl.when(j + 2 < trips)
                    def _():
                        @pl.when(j >= 2)
                        def _():
                            swait(bf)

                        gstart(j + 2, bf)

                    gwait(bb)
                    sstart(j, bb)
            return c

        lax.fori_loop(0, trips, step, 0)
        for bb in range(4):
            @pl.when(trips >= bb + 1)
            def _():
                swait(bb)

        _copy_out(stage_v, acc_sh, out_hbm, cid, sid, rpt)

    return k(table, src2, dst2)


_BLK = 1000


def _tc_mm(x, W1, n_pad):
    N, F_IN = x.shape

    def body(x_ref, w_ref, hx_ref):
        hx_ref[...] = jnp.dot(x_ref[...], w_ref[...],
                              preferred_element_type=jnp.float32)

    return pl.pallas_call(
        body,
        grid=(N // _BLK,),
        in_specs=[
            pl.BlockSpec((_BLK, F_IN), lambda i: (i, 0)),
            pl.BlockSpec((F_IN, _F), lambda i: (0, 0)),
        ],
        out_specs=pl.BlockSpec((_BLK, _F), lambda i: (i, 0)),
        out_shape=jax.ShapeDtypeStruct((n_pad, _F), jnp.float32),
    )(x, W1)


def _tc_prep(hx, cnt_p, N):
    n_pad = hx.shape[0]

    def body(hx_ref, cnt_ref, hxs_ref, dinvb_ref, invcb_ref):
        cnt = cnt_ref[0] + cnt_ref[1]
        dinvb = lax.rsqrt(cnt + 1.0)
        dinvb_ref[...] = dinvb
        invcb_ref[...] = 1.0 / jnp.maximum(cnt, 1.0)
        hxs_ref[...] = hx_ref[...] * dinvb

    shp = jax.ShapeDtypeStruct((n_pad, _F), jnp.float32)
    blk = pl.BlockSpec((_BLK, _F), lambda i: (i, 0))
    return pl.pallas_call(
        body,
        grid=(N // _BLK,),
        in_specs=[blk, pl.BlockSpec((_NC, _BLK, _F), lambda i: (0, i, 0))],
        out_specs=(blk, blk, blk),
        out_shape=(shp, shp, shp),
    )(hx, cnt_p)


def _tc_comb(t1_p, hx, dinvb, b1, N):
    n_pad = hx.shape[0]

    def body(t1_ref, hx_ref, dinvb_ref, b1_ref, h_ref):
        d = dinvb_ref[...]
        t1 = t1_ref[0] + t1_ref[1]
        h_ref[...] = d * t1 + d * d * hx_ref[...] + b1_ref[...][None, :]

    blk = pl.BlockSpec((_BLK, _F), lambda i: (i, 0))
    return pl.pallas_call(
        body,
        grid=(N // _BLK,),
        in_specs=[pl.BlockSpec((_NC, _BLK, _F), lambda i: (0, i, 0)), blk, blk,
                  pl.BlockSpec((_F,), lambda i: (0,))],
        out_specs=blk,
        out_shape=jax.ShapeDtypeStruct((n_pad, _F), jnp.float32),
    )(t1_p, hx, dinvb, b1)


def _tc_out(t2_p, h, invcb, Wl, bl, Wr, N):
    C = Wl.shape[1]

    def body(t2_ref, h_ref, invcb_ref, wl_ref, bl_ref, wr_ref, o_ref):
        mean = (t2_ref[0] + t2_ref[1]) * invcb_ref[...]
        hh = h_ref[...]
        o = (jnp.dot(mean, wl_ref[...], preferred_element_type=jnp.float32)
             + jnp.dot(hh, wr_ref[...], preferred_element_type=jnp.float32)
             + bl_ref[...][None, :])
        m = jnp.max(o, axis=1, keepdims=True)
        lse = m + jnp.log(jnp.sum(jnp.exp(o - m), axis=1, keepdims=True))
        o_ref[...] = o - lse

    blk = pl.BlockSpec((_BLK, _F), lambda i: (i, 0))
    return pl.pallas_call(
        body,
        grid=(N // _BLK,),
        in_specs=[
            pl.BlockSpec((_NC, _BLK, _F), lambda i: (0, i, 0)),
            blk,
            blk,
            pl.BlockSpec((_F, C), lambda i: (0, 0)),
            pl.BlockSpec((C,), lambda i: (0,)),
            pl.BlockSpec((_F, C), lambda i: (0, 0)),
        ],
        out_specs=pl.BlockSpec((_BLK, C), lambda i: (i, 0)),
        out_shape=jax.ShapeDtypeStruct((N, C), jnp.float32),
    )(t2_p, h, invcb, Wl, bl, Wr)


def kernel(x, edge_index, W1, b1, Wl, bl, Wr):
    N = x.shape[0]
    E = edge_index.shape[1]
    assert E % _CHUNK == 0 and N % _NS == 0 and N % _BLK == 0
    nchunks = E // _CHUNK
    assert (_NW - 1) * _TPW <= nchunks <= _NW * _TPW
    src2 = edge_index[0].reshape(nchunks, _CHUNK)
    dst2 = edge_index[1].reshape(nchunks, _CHUNK)

    hx = _tc_mm(x, W1, N)
    cnt_p = _sc_count(dst2, N)
    hxs, dinvb, invcb = _tc_prep(hx, cnt_p, N)
    t1_p = _sc_segsum(hxs, src2, dst2, N)
    h = _tc_comb(t1_p, hx, dinvb, b1, N)
    t2_p = _sc_segsum(h, src2, dst2, N)
    return _tc_out(t2_p, h, invcb, Wl, bl, Wr, N)


# speedup vs baseline: 1.4814x; 1.3285x over previous
import functools

import jax
import jax.numpy as jnp
from jax import lax
from jax.experimental import pallas as pl
from jax.experimental.pallas import tpu as pltpu
from jax.experimental.pallas import tpu_sc as plsc

_NC = 2
_NS = 16
_NW = _NC * _NS
_CHUNK = 128
_F = 16
_TPW = 80


def _mesh():
    return plsc.VectorSubcoreMesh(core_axis_name="c", subcore_axis_name="s")


def _zero_acc(stage_v, acc_sh, sid, rpt):
    def fill_zero(i, c):
        stage_v[i] = jnp.zeros((_F,), jnp.float32)
        return c

    lax.fori_loop(0, rpt, fill_zero, 0)
    pltpu.sync_copy(stage_v, acc_sh.at[pl.ds(sid * rpt, rpt)])
    plsc.subcore_barrier()


def _copy_out(stage_v, acc_sh, out_hbm, cid, sid, rpt):
    plsc.subcore_barrier()
    pltpu.sync_copy(acc_sh.at[pl.ds(sid * rpt, rpt)], stage_v)
    pltpu.sync_copy(stage_v, out_hbm.at[cid, pl.ds(sid * rpt, rpt)])


def _tile_span(wid, nchunks):
    last = nchunks - (_NW - 1) * _TPW
    base = wid * _TPW
    trips = jnp.where(wid < _NW - 1, _TPW, last)
    return base, trips


def _load_idx(idx_hbm, idx_v, wid, base, nchunks):
    last = nchunks - (_NW - 1) * _TPW

    @pl.when(wid < _NW - 1)
    def _():
        pltpu.sync_copy(idx_hbm.at[pl.ds(base, _TPW), :], idx_v)

    if last < _TPW:
        @pl.when(wid == _NW - 1)
        def _():
            pltpu.sync_copy(idx_hbm.at[pl.ds((_NW - 1) * _TPW, last), :],
                            idx_v.at[pl.ds(0, last), :])


def _sc_count(dst2, n):
    nchunks = dst2.shape[0]
    rpt = n // _NS

    @functools.partial(
        pl.kernel,
        mesh=_mesh(),
        out_type=jax.ShapeDtypeStruct((_NC, n, _F), jnp.float32),
        compiler_params=pltpu.CompilerParams(use_tc_tiling_on_sc=False),
        scratch_types=[
            pltpu.VMEM((_TPW, _CHUNK), jnp.int32),
            pltpu.VMEM((_CHUNK, _F), jnp.float32),
            pltpu.VMEM((rpt, _F), jnp.float32),
            pltpu.VMEM_SHARED((n, _F), jnp.float32),
            pltpu.SemaphoreType.DMA,
            pltpu.SemaphoreType.DMA,
            pltpu.SemaphoreType.DMA,
            pltpu.SemaphoreType.DMA,
        ],
    )
    def k(dst_hbm, out_hbm, didx_v, ones_v, stage_v, acc_sh, s0, s1, s2, s3):
        cid = lax.axis_index("c")
        sid = lax.axis_index("s")
        wid = sid * _NC + cid
        base, trips = _tile_span(wid, nchunks)
        sems = [s0, s1, s2, s3]

        def fill_ones(i, c):
            ones_v[i] = jnp.ones((_F,), jnp.float32)
            return c

        lax.fori_loop(0, _CHUNK, fill_ones, 0)
        _load_idx(dst_hbm, didx_v, wid, base, nchunks)
        _zero_acc(stage_v, acc_sh, sid, rpt)

        def swait(sem):
            pltpu.make_async_copy(ones_v, acc_sh.at[didx_v.at[0]], sem).wait()

        def step(j, c):
            b = lax.rem(j, 4)

            @pl.when(j >= 4)
            def _():
                for bb in range(4):
                    @pl.when(b == bb)
                    def _():
                        swait(sems[bb])

            for bb in range(4):
                @pl.when(b == bb)
                def _():
                    pltpu.async_copy(ones_v, acc_sh.at[didx_v.at[j]],
                                     sems[bb], add=True)
            return c

        lax.fori_loop(0, trips, step, 0)
        for bb in range(4):
            @pl.when(trips >= bb + 1)
            def _():
                swait(sems[bb])

        _copy_out(stage_v, acc_sh, out_hbm, cid, sid, rpt)

    return k(dst2)


def _sc_segsum(table, src2, dst2, n):
    nchunks = src2.shape[0]
    rpt = n // _NS

    @functools.partial(
        pl.kernel,
        mesh=_mesh(),
        out_type=jax.ShapeDtypeStruct((_NC, n, _F), jnp.float32),
        compiler_params=pltpu.CompilerParams(use_tc_tiling_on_sc=False),
        scratch_types=[
            pltpu.VMEM((_TPW, _CHUNK), jnp.int32),
            pltpu.VMEM((_TPW, _CHUNK), jnp.int32),
            pltpu.VMEM((_CHUNK, _F), jnp.float32),
            pltpu.VMEM((_CHUNK, _F), jnp.float32),
            pltpu.VMEM((_CHUNK, _F), jnp.float32),
            pltpu.VMEM((_CHUNK, _F), jnp.float32),
            pltpu.VMEM((rpt, _F), jnp.float32),
            pltpu.VMEM_SHARED((n, _F), jnp.float32),
            pltpu.VMEM_SHARED((n, _F), jnp.float32),
            pltpu.SemaphoreType.DMA,
            pltpu.SemaphoreType.DMA,
            pltpu.SemaphoreType.DMA,
            pltpu.SemaphoreType.DMA,
            pltpu.SemaphoreType.DMA,
            pltpu.SemaphoreType.DMA,
            pltpu.SemaphoreType.DMA,
            pltpu.SemaphoreType.DMA,
        ],
    )
    def k(table_hbm, src_hbm, dst_hbm, out_hbm,
          sidx_v, didx_v, r0, r1, r2, r3, stage_v, tab_sh, acc_sh,
          g0, g1, g2, g3, s0, s1, s2, s3):
        cid = lax.axis_index("c")
        sid = lax.axis_index("s")
        wid = sid * _NC + cid
        base, trips = _tile_span(wid, nchunks)
        rows = [r0, r1, r2, r3]
        gsem = [g0, g1, g2, g3]
        ssem = [s0, s1, s2, s3]

        _load_idx(src_hbm, sidx_v, wid, base, nchunks)
        _load_idx(dst_hbm, didx_v, wid, base, nchunks)
        pltpu.sync_copy(table_hbm.at[pl.ds(sid * rpt, rpt), :], stage_v)
        pltpu.sync_copy(stage_v, tab_sh.at[pl.ds(sid * rpt, rpt)])
        _zero_acc(stage_v, acc_sh, sid, rpt)

        def gstart(j, b):
            pltpu.async_copy(tab_sh.at[sidx_v.at[j]], rows[b], gsem[b])

        def gwait(b):
            pltpu.make_async_copy(tab_sh.at[sidx_v.at[0]], rows[b],
                                  gsem[b]).wait()

        def sstart(j, b):
            pltpu.async_copy(rows[b], acc_sh.at[didx_v.at[j]], ssem[b],
                             add=True)

        def swait(b):
            pltpu.make_async_copy(rows[b], acc_sh.at[didx_v.at[0]],
                                  ssem[b]).wait()

        @pl.when(trips >= 1)
        def _():
            gstart(0, 0)

        @pl.when(trips >= 2)
        def _():
            gstart(1, 1)

        def step(j, c):
            b = lax.rem(j, 4)
            for bb in range(4):
                @pl.when(b == bb)
                def _():
                    bf = (bb + 2) % 4

                    @pl.when(j + 2 < trips)
                    def _():
                        @pl.when(j >= 2)
                        def _():
                            swait(bf)

                        gstart(j + 2, bf)

                    gwait(bb)
                    sstart(j, bb)
            return c

        lax.fori_loop(0, trips, step, 0)
        for bb in range(4):
            @pl.when(trips >= bb + 1)
            def _():
                swait(bb)

        _copy_out(stage_v, acc_sh, out_hbm, cid, sid, rpt)

    return k(table, src2, dst2)


_G = 8
_PBLK = 128


def _tc_mm(xv, W1bd, np_rows):
    K = xv.shape[1]

    def body(x_ref, w_ref, hx_ref):
        hx_ref[...] = jnp.dot(x_ref[...], w_ref[...],
                              preferred_element_type=jnp.float32)

    return pl.pallas_call(
        body,
        grid=(np_rows // _PBLK,),
        in_specs=[
            pl.BlockSpec((_PBLK, K), lambda i: (i, 0)),
            pl.BlockSpec((K, _G * _F), lambda i: (0, 0)),
        ],
        out_specs=pl.BlockSpec((_PBLK, _G * _F), lambda i: (i, 0)),
        out_shape=jax.ShapeDtypeStruct((np_rows, _G * _F), jnp.float32),
    )(xv, W1bd)


def _tc_prep(hxp, cnt_pp):
    np_rows = hxp.shape[0]

    def body(hx_ref, cnt_ref, hxs_ref, dinv_ref, invc_ref):
        cnt = cnt_ref[0] + cnt_ref[1]
        dinv = lax.rsqrt(cnt + 1.0)
        dinv_ref[...] = dinv
        invc_ref[...] = 1.0 / jnp.maximum(cnt, 1.0)
        hxs_ref[...] = hx_ref[...] * dinv

    shp = jax.ShapeDtypeStruct((np_rows, _G * _F), jnp.float32)
    blk = pl.BlockSpec((_PBLK, _G * _F), lambda i: (i, 0))
    return pl.pallas_call(
        body,
        grid=(np_rows // _PBLK,),
        in_specs=[blk, pl.BlockSpec((_NC, _PBLK, _G * _F), lambda i: (0, i, 0))],
        out_specs=(blk, blk, blk),
        out_shape=(shp, shp, shp),
    )(hxp, cnt_pp)


def _tc_comb(t1_pp, hxp, dinvp, b1t):
    np_rows = hxp.shape[0]

    def body(t1_ref, hx_ref, dinv_ref, b1_ref, h_ref):
        d = dinv_ref[...]
        t1 = t1_ref[0] + t1_ref[1]
        h_ref[...] = d * t1 + d * d * hx_ref[...] + b1_ref[...][None, :]

    blk = pl.BlockSpec((_PBLK, _G * _F), lambda i: (i, 0))
    return pl.pallas_call(
        body,
        grid=(np_rows // _PBLK,),
        in_specs=[pl.BlockSpec((_NC, _PBLK, _G * _F), lambda i: (0, i, 0)),
                  blk, blk, pl.BlockSpec((_G * _F,), lambda i: (0,))],
        out_specs=blk,
        out_shape=jax.ShapeDtypeStruct((np_rows, _G * _F), jnp.float32),
    )(t1_pp, hxp, dinvp, b1t)


def _tc_out(t2_pp, hp, invcp, Ewide, Wlt, blv, Wrt):
    np_rows = hp.shape[0]
    C = Wlt.shape[1]
    rblk = _PBLK * _G

    def body(t2_ref, h_ref, invc_ref, e_ref, wl_ref, bl_ref, wr_ref, o_ref):
        mean = (t2_ref[0] + t2_ref[1]) * invc_ref[...]
        e = e_ref[...]
        qm = jnp.dot(e, mean, preferred_element_type=jnp.float32)
        qh = jnp.dot(e, h_ref[...], preferred_element_type=jnp.float32)
        row = jax.lax.broadcasted_iota(jnp.int32, (rblk, _G * _F), 0)
        lane = jax.lax.broadcasted_iota(jnp.int32, (rblk, _G * _F), 1)
        mask = ((lane // _F) == (row % _G)).astype(jnp.float32)
        o = (jnp.dot(qm * mask, wl_ref[...], preferred_element_type=jnp.float32)
             + jnp.dot(qh * mask, wr_ref[...], preferred_element_type=jnp.float32)
             + bl_ref[...][None, :])
        m = jnp.max(o, axis=1, keepdims=True)
        lse = m + jnp.log(jnp.sum(jnp.exp(o - m), axis=1, keepdims=True))
        o_ref[...] = o - lse

    blk = pl.BlockSpec((_PBLK, _G * _F), lambda i: (i, 0))
    return pl.pallas_call(
        body,
        grid=(np_rows // _PBLK,),
        in_specs=[
            pl.BlockSpec((_NC, _PBLK, _G * _F), lambda i: (0, i, 0)),
            blk,
            blk,
            pl.BlockSpec((rblk, _PBLK), lambda i: (0, 0)),
            pl.BlockSpec((_G * _F, C), lambda i: (0, 0)),
            pl.BlockSpec((C,), lambda i: (0,)),
            pl.BlockSpec((_G * _F, C), lambda i: (0, 0)),
        ],
        out_specs=pl.BlockSpec((rblk, C), lambda i: (i, 0)),
        out_shape=jax.ShapeDtypeStruct((np_rows * _G, C), jnp.float32),
    )(t2_pp, hp, invcp, Ewide, Wlt, blv, Wrt)


def kernel(x, edge_index, W1, b1, Wl, bl, Wr):
    N, F_IN = x.shape
    E = edge_index.shape[1]
    n_pad = ((N + _G * _PBLK - 1) // (_G * _PBLK)) * (_G * _PBLK)
    np_rows = n_pad // _G
    assert E % _CHUNK == 0 and np_rows % _PBLK == 0
    nchunks = E // _CHUNK
    assert (_NW - 1) * _TPW <= nchunks <= _NW * _TPW
    src2 = edge_index[0].reshape(nchunks, _CHUNK)
    dst2 = edge_index[1].reshape(nchunks, _CHUNK)

    xv = jnp.pad(x, ((0, n_pad - N), (0, 0))).reshape(np_rows, _G * F_IN)
    W1bd = jnp.einsum("ab,kf->akbf", jnp.eye(_G, dtype=x.dtype),
                      W1).reshape(_G * F_IN, _G * _F)
    b1t = jnp.tile(b1, _G)
    Wlt = jnp.tile(Wl, (_G, 1))
    Wrt = jnp.tile(Wr, (_G, 1))
    rblk = _PBLK * _G
    Ewide = (jax.lax.broadcasted_iota(jnp.int32, (rblk, _PBLK), 0) // _G
             == jax.lax.broadcasted_iota(jnp.int32, (rblk, _PBLK), 1)
             ).astype(jnp.float32)

    hxp = _tc_mm(xv, W1bd, np_rows)
    cnt_p = _sc_count(dst2, n_pad)
    cnt_pp = cnt_p.reshape(_NC, np_rows, _G * _F)
    hxsp, dinvp, invcp = _tc_prep(hxp, cnt_pp)
    t1_p = _sc_segsum(hxsp.reshape(n_pad, _F), src2, dst2, n_pad)
    hp = _tc_comb(t1_p.reshape(_NC, np_rows, _G * _F), hxp, dinvp, b1t)
    t2_p = _sc_segsum(hp.reshape(n_pad, _F), src2, dst2, n_pad)
    out = _tc_out(t2_p.reshape(_NC, np_rows, _G * _F), hp, invcp,
                  Ewide, Wlt, bl, Wrt)
    return out[:N]


# speedup vs baseline: 1.6002x; 1.0802x over previous
import functools

import jax
import jax.numpy as jnp
from jax import lax
from jax.experimental import pallas as pl
from jax.experimental.pallas import tpu as pltpu
from jax.experimental.pallas import tpu_sc as plsc

_NC = 2
_NS = 16
_NW = _NC * _NS
_CHUNK = 128
_F = 16
_TPW = 80


def _mesh():
    return plsc.VectorSubcoreMesh(core_axis_name="c", subcore_axis_name="s")


def _zero_acc(stage_v, acc_sh, sid, rpt):
    def fill_zero(i, c):
        stage_v[i] = jnp.zeros((_F,), jnp.float32)
        return c

    lax.fori_loop(0, rpt, fill_zero, 0)
    pltpu.sync_copy(stage_v, acc_sh.at[pl.ds(sid * rpt, rpt)])
    plsc.subcore_barrier()


def _copy_out(stage_v, acc_sh, out_hbm, cid, sid, rpt):
    plsc.subcore_barrier()
    pltpu.sync_copy(acc_sh.at[pl.ds(sid * rpt, rpt)], stage_v)
    pltpu.sync_copy(stage_v, out_hbm.at[cid, pl.ds(sid * rpt, rpt)])


def _tile_span(wid, nchunks):
    last = nchunks - (_NW - 1) * _TPW
    base = wid * _TPW
    trips = jnp.where(wid < _NW - 1, _TPW, last)
    return base, trips


def _load_idx(idx_hbm, idx_v, wid, base, nchunks):
    last = nchunks - (_NW - 1) * _TPW

    @pl.when(wid < _NW - 1)
    def _():
        pltpu.sync_copy(idx_hbm.at[pl.ds(base, _TPW), :], idx_v)

    if last < _TPW:
        @pl.when(wid == _NW - 1)
        def _():
            pltpu.sync_copy(idx_hbm.at[pl.ds((_NW - 1) * _TPW, last), :],
                            idx_v.at[pl.ds(0, last), :])


def _sc_count(dst2, n, nchunks):
    rpt = n // _NS

    @functools.partial(
        pl.kernel,
        mesh=_mesh(),
        out_type=jax.ShapeDtypeStruct((_NC, n, _F), jnp.float32),
        compiler_params=pltpu.CompilerParams(use_tc_tiling_on_sc=False),
        scratch_types=[
            pltpu.VMEM((_TPW, _CHUNK), jnp.int32),
            pltpu.VMEM((_CHUNK, _F), jnp.float32),
            pltpu.VMEM((rpt, _F), jnp.float32),
            pltpu.VMEM_SHARED((n, _F), jnp.float32),
            pltpu.SemaphoreType.DMA,
            pltpu.SemaphoreType.DMA,
            pltpu.SemaphoreType.DMA,
            pltpu.SemaphoreType.DMA,
        ],
    )
    def k(dst_hbm, out_hbm, didx_v, ones_v, stage_v, acc_sh, s0, s1, s2, s3):
        cid = lax.axis_index("c")
        sid = lax.axis_index("s")
        wid = sid * _NC + cid
        base, trips = _tile_span(wid, nchunks)
        sems = [s0, s1, s2, s3]

        def fill_ones(i, c):
            ones_v[i] = jnp.ones((_F,), jnp.float32)
            return c

        lax.fori_loop(0, _CHUNK, fill_ones, 0)
        _load_idx(dst_hbm, didx_v, wid, base, nchunks)
        _zero_acc(stage_v, acc_sh, sid, rpt)

        def swait(sem):
            pltpu.make_async_copy(ones_v, acc_sh.at[didx_v.at[0]], sem).wait()

        def step(j, c):
            b = lax.rem(j, 4)

            @pl.when(j >= 4)
            def _():
                for bb in range(4):
                    @pl.when(b == bb)
                    def _():
                        swait(sems[bb])

            for bb in range(4):
                @pl.when(b == bb)
                def _():
                    pltpu.async_copy(ones_v, acc_sh.at[didx_v.at[j]],
                                     sems[bb], add=True)
            return c

        lax.fori_loop(0, trips, step, 0)
        for bb in range(4):
            @pl.when(trips >= bb + 1)
            def _():
                swait(sems[bb])

        _copy_out(stage_v, acc_sh, out_hbm, cid, sid, rpt)

    return k(dst2)


def _sc_segsum(table, src2, dst2, n, nchunks):
    rpt = n // _NS

    @functools.partial(
        pl.kernel,
        mesh=_mesh(),
        out_type=jax.ShapeDtypeStruct((_NC, n, _F), jnp.float32),
        compiler_params=pltpu.CompilerParams(use_tc_tiling_on_sc=False),
        scratch_types=[
            pltpu.VMEM((_TPW, _CHUNK), jnp.int32),
            pltpu.VMEM((_TPW, _CHUNK), jnp.int32),
            pltpu.VMEM((_CHUNK, _F), jnp.float32),
            pltpu.VMEM((_CHUNK, _F), jnp.float32),
            pltpu.VMEM((_CHUNK, _F), jnp.float32),
            pltpu.VMEM((_CHUNK, _F), jnp.float32),
            pltpu.VMEM((rpt, _F), jnp.float32),
            pltpu.VMEM_SHARED((n, _F), jnp.float32),
            pltpu.VMEM_SHARED((n, _F), jnp.float32),
            pltpu.SemaphoreType.DMA,
            pltpu.SemaphoreType.DMA,
            pltpu.SemaphoreType.DMA,
            pltpu.SemaphoreType.DMA,
            pltpu.SemaphoreType.DMA,
            pltpu.SemaphoreType.DMA,
            pltpu.SemaphoreType.DMA,
            pltpu.SemaphoreType.DMA,
        ],
    )
    def k(table_hbm, src_hbm, dst_hbm, out_hbm,
          sidx_v, didx_v, r0, r1, r2, r3, stage_v, tab_sh, acc_sh,
          g0, g1, g2, g3, s0, s1, s2, s3):
        cid = lax.axis_index("c")
        sid = lax.axis_index("s")
        wid = sid * _NC + cid
        base, trips = _tile_span(wid, nchunks)
        rows = [r0, r1, r2, r3]
        gsem = [g0, g1, g2, g3]
        ssem = [s0, s1, s2, s3]

        _load_idx(src_hbm, sidx_v, wid, base, nchunks)
        _load_idx(dst_hbm, didx_v, wid, base, nchunks)
        pltpu.sync_copy(table_hbm.at[pl.ds(sid * rpt, rpt), :], stage_v)
        pltpu.sync_copy(stage_v, tab_sh.at[pl.ds(sid * rpt, rpt)])
        _zero_acc(stage_v, acc_sh, sid, rpt)

        def gstart(j, b):
            pltpu.async_copy(tab_sh.at[sidx_v.at[j]], rows[b], gsem[b])

        def gwait(b):
            pltpu.make_async_copy(tab_sh.at[sidx_v.at[0]], rows[b],
                                  gsem[b]).wait()

        def sstart(j, b):
            pltpu.async_copy(rows[b], acc_sh.at[didx_v.at[j]], ssem[b],
                             add=True)

        def swait(b):
            pltpu.make_async_copy(rows[b], acc_sh.at[didx_v.at[0]],
                                  ssem[b]).wait()

        @pl.when(trips >= 1)
        def _():
            gstart(0, 0)

        @pl.when(trips >= 2)
        def _():
            gstart(1, 1)

        def step(j, c):
            b = lax.rem(j, 4)
            for bb in range(4):
                @pl.when(b == bb)
                def _():
                    bf = (bb + 2) % 4

                    @pl.when(j + 2 < trips)
                    def _():
                        @pl.when(j >= 2)
                        def _():
                            swait(bf)

                        gstart(j + 2, bf)

                    gwait(bb)
                    sstart(j, bb)
            return c

        lax.fori_loop(0, trips, step, 0)
        for bb in range(4):
            @pl.when(trips >= bb + 1)
            def _():
                swait(bb)

        _copy_out(stage_v, acc_sh, out_hbm, cid, sid, rpt)

    return k(table, src2, dst2)


_G = 8
_PBLK = 128


def _tc_edges(edge_index, nc_pad):
    blk_rows = 256
    nblk = nc_pad // blk_rows

    def body(e_ref, s_ref, d_ref):
        s_ref[...] = e_ref[0].reshape(blk_rows, _CHUNK)
        d_ref[...] = e_ref[1].reshape(blk_rows, _CHUNK)

    oblk = pl.BlockSpec((blk_rows, _CHUNK), lambda i: (i, 0))
    shp = jax.ShapeDtypeStruct((nc_pad, _CHUNK), jnp.int32)
    return pl.pallas_call(
        body,
        grid=(nblk,),
        in_specs=[pl.BlockSpec((2, blk_rows * _CHUNK), lambda i: (0, i))],
        out_specs=(oblk, oblk),
        out_shape=(shp, shp),
    )(edge_index)


def _tc_mm(xv, W1bd, np_rows):
    K = xv.shape[1]

    def body(x_ref, w_ref, hx_ref):
        hx_ref[...] = jnp.dot(x_ref[...], w_ref[...],
                              preferred_element_type=jnp.float32)

    return pl.pallas_call(
        body,
        grid=(np_rows // _PBLK,),
        in_specs=[
            pl.BlockSpec((_PBLK, K), lambda i: (i, 0)),
            pl.BlockSpec((K, _G * _F), lambda i: (0, 0)),
        ],
        out_specs=pl.BlockSpec((_PBLK, _G * _F), lambda i: (i, 0)),
        out_shape=jax.ShapeDtypeStruct((np_rows, _G * _F), jnp.float32),
    )(xv, W1bd)


def _tc_prep(hxp, cnt_pp):
    np_rows = hxp.shape[0]

    def body(hx_ref, cnt_ref, hxs_ref, dinv_ref, invc_ref):
        cnt = cnt_ref[0] + cnt_ref[1]
        dinv = lax.rsqrt(cnt + 1.0)
        dinv_ref[...] = dinv
        invc_ref[...] = 1.0 / jnp.maximum(cnt, 1.0)
        hxs_ref[...] = hx_ref[...] * dinv

    shp = jax.ShapeDtypeStruct((np_rows, _G * _F), jnp.float32)
    blk = pl.BlockSpec((_PBLK, _G * _F), lambda i: (i, 0))
    return pl.pallas_call(
        body,
        grid=(np_rows // _PBLK,),
        in_specs=[blk, pl.BlockSpec((_NC, _PBLK, _G * _F), lambda i: (0, i, 0))],
        out_specs=(blk, blk, blk),
        out_shape=(shp, shp, shp),
    )(hxp, cnt_pp)


def _tc_comb(t1_pp, hxp, dinvp, b1t):
    np_rows = hxp.shape[0]

    def body(t1_ref, hx_ref, dinv_ref, b1_ref, h_ref):
        d = dinv_ref[...]
        t1 = t1_ref[0] + t1_ref[1]
        h_ref[...] = d * t1 + d * d * hx_ref[...] + b1_ref[...][None, :]

    blk = pl.BlockSpec((_PBLK, _G * _F), lambda i: (i, 0))
    return pl.pallas_call(
        body,
        grid=(np_rows // _PBLK,),
        in_specs=[pl.BlockSpec((_NC, _PBLK, _G * _F), lambda i: (0, i, 0)),
                  blk, blk, pl.BlockSpec((_G * _F,), lambda i: (0,))],
        out_specs=blk,
        out_shape=jax.ShapeDtypeStruct((np_rows, _G * _F), jnp.float32),
    )(t1_pp, hxp, dinvp, b1t)


def _tc_out(t2_pp, hp, invcp, Ewide, Wlt, blv, Wrt):
    np_rows = hp.shape[0]
    C = Wlt.shape[1]
    rblk = _PBLK * _G

    def body(t2_ref, h_ref, invc_ref, e_ref, wl_ref, bl_ref, wr_ref, o_ref):
        mean = (t2_ref[0] + t2_ref[1]) * invc_ref[...]
        e = e_ref[...]
        qm = jnp.dot(e, mean, preferred_element_type=jnp.float32)
        qh = jnp.dot(e, h_ref[...], preferred_element_type=jnp.float32)
        row = jax.lax.broadcasted_iota(jnp.int32, (rblk, _G * _F), 0)
        lane = jax.lax.broadcasted_iota(jnp.int32, (rblk, _G * _F), 1)
        mask = ((lane // _F) == (row % _G)).astype(jnp.float32)
        o = (jnp.dot(qm * mask, wl_ref[...], preferred_element_type=jnp.float32)
             + jnp.dot(qh * mask, wr_ref[...], preferred_element_type=jnp.float32)
             + bl_ref[...][None, :])
        m = jnp.max(o, axis=1, keepdims=True)
        lse = m + jnp.log(jnp.sum(jnp.exp(o - m), axis=1, keepdims=True))
        o_ref[...] = o - lse

    blk = pl.BlockSpec((_PBLK, _G * _F), lambda i: (i, 0))
    return pl.pallas_call(
        body,
        grid=(np_rows // _PBLK,),
        in_specs=[
            pl.BlockSpec((_NC, _PBLK, _G * _F), lambda i: (0, i, 0)),
            blk,
            blk,
            pl.BlockSpec((rblk, _PBLK), lambda i: (0, 0)),
            pl.BlockSpec((_G * _F, C), lambda i: (0, 0)),
            pl.BlockSpec((C,), lambda i: (0,)),
            pl.BlockSpec((_G * _F, C), lambda i: (0, 0)),
        ],
        out_specs=pl.BlockSpec((rblk, C), lambda i: (i, 0)),
        out_shape=jax.ShapeDtypeStruct((np_rows * _G, C), jnp.float32),
    )(t2_pp, hp, invcp, Ewide, Wlt, blv, Wrt)


def kernel(x, edge_index, W1, b1, Wl, bl, Wr):
    N, F_IN = x.shape
    E = edge_index.shape[1]
    n_pad = ((N + _G * _PBLK - 1) // (_G * _PBLK)) * (_G * _PBLK)
    np_rows = n_pad // _G
    assert E % _CHUNK == 0 and np_rows % _PBLK == 0
    nchunks = E // _CHUNK
    assert (_NW - 1) * _TPW <= nchunks <= _NW * _TPW
    nc_pad = ((nchunks + 255) // 256) * 256
    src2, dst2 = _tc_edges(edge_index, nc_pad)

    xv = jnp.pad(x, ((0, n_pad - N), (0, 0))).reshape(np_rows, _G * F_IN)
    W1bd = jnp.einsum("ab,kf->akbf", jnp.eye(_G, dtype=x.dtype),
                      W1).reshape(_G * F_IN, _G * _F)
    b1t = jnp.tile(b1, _G)
    Wlt = jnp.tile(Wl, (_G, 1))
    Wrt = jnp.tile(Wr, (_G, 1))
    rblk = _PBLK * _G
    Ewide = (jax.lax.broadcasted_iota(jnp.int32, (rblk, _PBLK), 0) // _G
             == jax.lax.broadcasted_iota(jnp.int32, (rblk, _PBLK), 1)
             ).astype(jnp.float32)

    hxp = _tc_mm(xv, W1bd, np_rows)
    cnt_p = _sc_count(dst2, n_pad, nchunks)
    cnt_pp = cnt_p.reshape(_NC, np_rows, _G * _F)
    hxsp, dinvp, invcp = _tc_prep(hxp, cnt_pp)
    t1_p = _sc_segsum(hxsp.reshape(n_pad, _F), src2, dst2, n_pad, nchunks)
    hp = _tc_comb(t1_p.reshape(_NC, np_rows, _G * _F), hxp, dinvp, b1t)
    t2_p = _sc_segsum(hp.reshape(n_pad, _F), src2, dst2, n_pad, nchunks)
    out = _tc_out(t2_p.reshape(_NC, np_rows, _G * _F), hp, invcp,
                  Ewide, Wlt, bl, Wrt)
    return out[:N]


# speedup vs baseline: 1.6112x; 1.0069x over previous
import functools

import jax
import jax.numpy as jnp
from jax import lax
from jax.experimental import pallas as pl
from jax.experimental.pallas import tpu as pltpu
from jax.experimental.pallas import tpu_sc as plsc

_NC = 2
_NS = 16
_NW = _NC * _NS
_CHUNK = 128
_F = 16
_TPW = 80


def _mesh():
    return plsc.VectorSubcoreMesh(core_axis_name="c", subcore_axis_name="s")


def _zero_acc(stage_v, acc_sh, sid, rpt):
    def fill_zero(i, c):
        stage_v[i] = jnp.zeros((_F,), jnp.float32)
        return c

    lax.fori_loop(0, rpt, fill_zero, 0)
    pltpu.sync_copy(stage_v, acc_sh.at[pl.ds(sid * rpt, rpt)])
    plsc.subcore_barrier()


def _copy_out(stage_v, acc_sh, out_hbm, cid, sid, rpt):
    plsc.subcore_barrier()
    pltpu.sync_copy(acc_sh.at[pl.ds(sid * rpt, rpt)], stage_v)
    pltpu.sync_copy(stage_v, out_hbm.at[cid, pl.ds(sid * rpt, rpt)])


def _tile_span(wid, nchunks):
    last = nchunks - (_NW - 1) * _TPW
    base = wid * _TPW
    trips = jnp.where(wid < _NW - 1, _TPW, last)
    return base, trips


def _load_idx(idx_hbm, idx_v, wid, base, nchunks):
    last = nchunks - (_NW - 1) * _TPW

    @pl.when(wid < _NW - 1)
    def _():
        pltpu.sync_copy(idx_hbm.at[pl.ds(base, _TPW), :], idx_v)

    if last < _TPW:
        @pl.when(wid == _NW - 1)
        def _():
            pltpu.sync_copy(idx_hbm.at[pl.ds((_NW - 1) * _TPW, last), :],
                            idx_v.at[pl.ds(0, last), :])


def _sc_count(dst2, n, nchunks):
    rpt = n // _NS

    @functools.partial(
        pl.kernel,
        mesh=_mesh(),
        out_type=jax.ShapeDtypeStruct((_NC, n, _F), jnp.float32),
        compiler_params=pltpu.CompilerParams(use_tc_tiling_on_sc=False),
        scratch_types=[
            pltpu.VMEM((_TPW, _CHUNK), jnp.int32),
            pltpu.VMEM((_CHUNK, _F), jnp.float32),
            pltpu.VMEM((rpt, _F), jnp.float32),
            pltpu.VMEM_SHARED((n, _F), jnp.float32),
            pltpu.SemaphoreType.DMA,
            pltpu.SemaphoreType.DMA,
            pltpu.SemaphoreType.DMA,
            pltpu.SemaphoreType.DMA,
        ],
    )
    def k(dst_hbm, out_hbm, didx_v, ones_v, stage_v, acc_sh, s0, s1, s2, s3):
        cid = lax.axis_index("c")
        sid = lax.axis_index("s")
        wid = sid * _NC + cid
        base, trips = _tile_span(wid, nchunks)
        sems = [s0, s1, s2, s3]

        def fill_ones(i, c):
            ones_v[i] = jnp.ones((_F,), jnp.float32)
            return c

        lax.fori_loop(0, _CHUNK, fill_ones, 0)
        _load_idx(dst_hbm, didx_v, wid, base, nchunks)
        _zero_acc(stage_v, acc_sh, sid, rpt)

        def swait(sem):
            pltpu.make_async_copy(ones_v, acc_sh.at[didx_v.at[0]], sem).wait()

        def step(j, c):
            b = lax.rem(j, 4)

            @pl.when(j >= 4)
            def _():
                for bb in range(4):
                    @pl.when(b == bb)
                    def _():
                        swait(sems[bb])

            for bb in range(4):
                @pl.when(b == bb)
                def _():
                    pltpu.async_copy(ones_v, acc_sh.at[didx_v.at[j]],
                                     sems[bb], add=True)
            return c

        lax.fori_loop(0, trips, step, 0)
        for bb in range(4):
            @pl.when(trips >= bb + 1)
            def _():
                swait(sems[bb])

        _copy_out(stage_v, acc_sh, out_hbm, cid, sid, rpt)

    return k(dst2)


def _sc_segsum(table, src2, dst2, n, nchunks):
    rpt = n // _NS

    @functools.partial(
        pl.kernel,
        mesh=_mesh(),
        out_type=jax.ShapeDtypeStruct((_NC, n, _F), jnp.float32),
        compiler_params=pltpu.CompilerParams(use_tc_tiling_on_sc=False),
        scratch_types=[
            pltpu.VMEM((_TPW, _CHUNK), jnp.int32),
            pltpu.VMEM((_TPW, _CHUNK), jnp.int32),
            pltpu.VMEM((_CHUNK, _F), jnp.float32),
            pltpu.VMEM((_CHUNK, _F), jnp.float32),
            pltpu.VMEM((_CHUNK, _F), jnp.float32),
            pltpu.VMEM((_CHUNK, _F), jnp.float32),
            pltpu.VMEM((_CHUNK, _F), jnp.float32),
            pltpu.VMEM((_CHUNK, _F), jnp.float32),
            pltpu.VMEM((rpt, _F), jnp.float32),
            pltpu.VMEM_SHARED((n, _F), jnp.float32),
            pltpu.VMEM_SHARED((n, _F), jnp.float32),
        ] + [pltpu.SemaphoreType.DMA] * 12,
    )
    def k(table_hbm, src_hbm, dst_hbm, out_hbm,
          sidx_v, didx_v, r0, r1, r2, r3, r4, r5, stage_v, tab_sh, acc_sh,
          *sems):
        cid = lax.axis_index("c")
        sid = lax.axis_index("s")
        wid = sid * _NC + cid
        base, trips = _tile_span(wid, nchunks)
        rows = [r0, r1, r2, r3, r4, r5]
        gsem = list(sems[:6])
        ssem = list(sems[6:])
        NB, DEPTH = 6, 3

        _load_idx(src_hbm, sidx_v, wid, base, nchunks)
        _load_idx(dst_hbm, didx_v, wid, base, nchunks)
        pltpu.sync_copy(table_hbm.at[pl.ds(sid * rpt, rpt), :], stage_v)
        pltpu.sync_copy(stage_v, tab_sh.at[pl.ds(sid * rpt, rpt)])
        _zero_acc(stage_v, acc_sh, sid, rpt)

        def gstart(j, b):
            pltpu.async_copy(tab_sh.at[sidx_v.at[j]], rows[b], gsem[b])

        def gwait(b):
            pltpu.make_async_copy(tab_sh.at[sidx_v.at[0]], rows[b],
                                  gsem[b]).wait()

        def sstart(j, b):
            pltpu.async_copy(rows[b], acc_sh.at[didx_v.at[j]], ssem[b],
                             add=True)

        def swait(b):
            pltpu.make_async_copy(rows[b], acc_sh.at[didx_v.at[0]],
                                  ssem[b]).wait()

        for p in range(DEPTH):
            @pl.when(trips >= p + 1)
            def _():
                gstart(p, p)

        def step(j, c):
            b = lax.rem(j, NB)
            for bb in range(NB):
                @pl.when(b == bb)
                def _():
                    bf = (bb + DEPTH) % NB

                    @pl.when(j + DEPTH < trips)
                    def _():
                        @pl.when(j >= NB - DEPTH)
                        def _():
                            swait(bf)

                        gstart(j + DEPTH, bf)

                    gwait(bb)
                    sstart(j, bb)
            return c

        lax.fori_loop(0, trips, step, 0)
        for bb in range(NB):
            @pl.when(trips >= bb + 1)
            def _():
                swait(bb)

        _copy_out(stage_v, acc_sh, out_hbm, cid, sid, rpt)

    return k(table, src2, dst2)


_G = 8
_PBLK = 128


def _tc_edges(edge_index, nc_pad):
    blk_rows = 256
    nblk = nc_pad // blk_rows

    def body(e_ref, s_ref, d_ref):
        s_ref[...] = e_ref[0].reshape(blk_rows, _CHUNK)
        d_ref[...] = e_ref[1].reshape(blk_rows, _CHUNK)

    oblk = pl.BlockSpec((blk_rows, _CHUNK), lambda i: (i, 0))
    shp = jax.ShapeDtypeStruct((nc_pad, _CHUNK), jnp.int32)
    return pl.pallas_call(
        body,
        grid=(nblk,),
        in_specs=[pl.BlockSpec((2, blk_rows * _CHUNK), lambda i: (0, i))],
        out_specs=(oblk, oblk),
        out_shape=(shp, shp),
    )(edge_index)


def _tc_mm(xv, W1bd, np_rows):
    K = xv.shape[1]

    def body(x_ref, w_ref, hx_ref):
        hx_ref[...] = jnp.dot(x_ref[...], w_ref[...],
                              preferred_element_type=jnp.float32)

    return pl.pallas_call(
        body,
        grid=(np_rows // _PBLK,),
        in_specs=[
            pl.BlockSpec((_PBLK, K), lambda i: (i, 0)),
            pl.BlockSpec((K, _G * _F), lambda i: (0, 0)),
        ],
        out_specs=pl.BlockSpec((_PBLK, _G * _F), lambda i: (i, 0)),
        out_shape=jax.ShapeDtypeStruct((np_rows, _G * _F), jnp.float32),
    )(xv, W1bd)


def _tc_prep(hxp, cnt_pp):
    np_rows = hxp.shape[0]

    def body(hx_ref, cnt_ref, hxs_ref, dinv_ref, invc_ref):
        cnt = cnt_ref[0] + cnt_ref[1]
        dinv = lax.rsqrt(cnt + 1.0)
        dinv_ref[...] = dinv
        invc_ref[...] = 1.0 / jnp.maximum(cnt, 1.0)
        hxs_ref[...] = hx_ref[...] * dinv

    shp = jax.ShapeDtypeStruct((np_rows, _G * _F), jnp.float32)
    blk = pl.BlockSpec((_PBLK, _G * _F), lambda i: (i, 0))
    return pl.pallas_call(
        body,
        grid=(np_rows // _PBLK,),
        in_specs=[blk, pl.BlockSpec((_NC, _PBLK, _G * _F), lambda i: (0, i, 0))],
        out_specs=(blk, blk, blk),
        out_shape=(shp, shp, shp),
    )(hxp, cnt_pp)


def _tc_comb(t1_pp, hxp, dinvp, b1t):
    np_rows = hxp.shape[0]

    def body(t1_ref, hx_ref, dinv_ref, b1_ref, h_ref):
        d = dinv_ref[...]
        t1 = t1_ref[0] + t1_ref[1]
        h_ref[...] = d * t1 + d * d * hx_ref[...] + b1_ref[...][None, :]

    blk = pl.BlockSpec((_PBLK, _G * _F), lambda i: (i, 0))
    return pl.pallas_call(
        body,
        grid=(np_rows // _PBLK,),
        in_specs=[pl.BlockSpec((_NC, _PBLK, _G * _F), lambda i: (0, i, 0)),
                  blk, blk, pl.BlockSpec((_G * _F,), lambda i: (0,))],
        out_specs=blk,
        out_shape=jax.ShapeDtypeStruct((np_rows, _G * _F), jnp.float32),
    )(t1_pp, hxp, dinvp, b1t)


def _tc_out(t2_pp, hp, invcp, Ewide, Wlt, blv, Wrt, N):
    np_rows = hp.shape[0]
    C = Wlt.shape[1]
    rblk = _PBLK * _G

    def body(t2_ref, h_ref, invc_ref, e_ref, wl_ref, bl_ref, wr_ref, o_ref):
        mean = (t2_ref[0] + t2_ref[1]) * invc_ref[...]
        e = e_ref[...]
        qm = jnp.dot(e, mean, preferred_element_type=jnp.float32)
        qh = jnp.dot(e, h_ref[...], preferred_element_type=jnp.float32)
        row = jax.lax.broadcasted_iota(jnp.int32, (rblk, _G * _F), 0)
        lane = jax.lax.broadcasted_iota(jnp.int32, (rblk, _G * _F), 1)
        mask = ((lane // _F) == (row % _G)).astype(jnp.float32)
        o = (jnp.dot(qm * mask, wl_ref[...], preferred_element_type=jnp.float32)
             + jnp.dot(qh * mask, wr_ref[...], preferred_element_type=jnp.float32)
             + bl_ref[...][None, :])
        m = jnp.max(o, axis=1, keepdims=True)
        lse = m + jnp.log(jnp.sum(jnp.exp(o - m), axis=1, keepdims=True))
        o_ref[...] = o - lse

    blk = pl.BlockSpec((_PBLK, _G * _F), lambda i: (i, 0))
    return pl.pallas_call(
        body,
        grid=(np_rows // _PBLK,),
        in_specs=[
            pl.BlockSpec((_NC, _PBLK, _G * _F), lambda i: (0, i, 0)),
            blk,
            blk,
            pl.BlockSpec((rblk, _PBLK), lambda i: (0, 0)),
            pl.BlockSpec((_G * _F, C), lambda i: (0, 0)),
            pl.BlockSpec((C,), lambda i: (0,)),
            pl.BlockSpec((_G * _F, C), lambda i: (0, 0)),
        ],
        out_specs=pl.BlockSpec((rblk, C), lambda i: (i, 0)),
        out_shape=jax.ShapeDtypeStruct((N, C), jnp.float32),
    )(t2_pp, hp, invcp, Ewide, Wlt, blv, Wrt)


def kernel(x, edge_index, W1, b1, Wl, bl, Wr):
    N, F_IN = x.shape
    E = edge_index.shape[1]
    n_pad = ((N + _G * _PBLK - 1) // (_G * _PBLK)) * (_G * _PBLK)
    np_rows = n_pad // _G
    assert E % _CHUNK == 0 and np_rows % _PBLK == 0
    nchunks = E // _CHUNK
    assert (_NW - 1) * _TPW <= nchunks <= _NW * _TPW
    nc_pad = ((nchunks + 255) // 256) * 256
    src2, dst2 = _tc_edges(edge_index, nc_pad)

    xv = jnp.pad(x, ((0, n_pad - N), (0, 0))).reshape(np_rows, _G * F_IN)
    W1bd = jnp.einsum("ab,kf->akbf", jnp.eye(_G, dtype=x.dtype),
                      W1).reshape(_G * F_IN, _G * _F)
    b1t = jnp.tile(b1, _G)
    Wlt = jnp.tile(Wl, (_G, 1))
    Wrt = jnp.tile(Wr, (_G, 1))
    rblk = _PBLK * _G
    Ewide = (jax.lax.broadcasted_iota(jnp.int32, (rblk, _PBLK), 0) // _G
             == jax.lax.broadcasted_iota(jnp.int32, (rblk, _PBLK), 1)
             ).astype(jnp.float32)

    hxp = _tc_mm(xv, W1bd, np_rows)
    cnt_p = _sc_count(dst2, n_pad, nchunks)
    cnt_pp = cnt_p.reshape(_NC, np_rows, _G * _F)
    hxsp, dinvp, invcp = _tc_prep(hxp, cnt_pp)
    t1_p = _sc_segsum(hxsp.reshape(n_pad, _F), src2, dst2, n_pad, nchunks)
    hp = _tc_comb(t1_p.reshape(_NC, np_rows, _G * _F), hxp, dinvp, b1t)
    t2_p = _sc_segsum(hp.reshape(n_pad, _F), src2, dst2, n_pad, nchunks)
    return _tc_out(t2_p.reshape(_NC, np_rows, _G * _F), hp, invcp,
                   Ewide, Wlt, bl, Wrt, N)
